# Initial kernel scaffold; baseline (speedup 1.0000x reference)
import jax
import jax.numpy as jnp
from jax.experimental import pallas as pl


def kernel(token_ids, weight):
    raise NotImplementedError("write your pallas kernel here")


# speedup vs baseline: 1.8553x; 1.8553x over previous
import functools

import jax
import jax.numpy as jnp
from jax import lax
from jax.experimental import pallas as pl
from jax.experimental.pallas import tpu as pltpu
from jax.experimental.pallas import tpu_sc as plsc

_NC = 2
_NS = 16
_NW = _NC * _NS

_CH = 128
_K = 8


def kernel(token_ids, weight):
    B0, S = token_ids.shape
    V, D = weight.shape
    B = B0 * S
    MC = _CH * _K
    b_per_w = B // _NW
    n_ch = b_per_w // _CH
    NM = b_per_w // MC
    assert B == _NW * n_ch * _CH and b_per_w % MC == 0

    idx = token_ids.reshape(_NW, n_ch, _CH).astype(jnp.int32)

    mesh = plsc.VectorSubcoreMesh(
        core_axis_name="c", subcore_axis_name="s",
        num_cores=_NC, num_subcores=_NS)

    @functools.partial(
        pl.kernel,
        out_type=jax.ShapeDtypeStruct((B, D), jnp.float32),
        mesh=mesh,
        compiler_params=pltpu.CompilerParams(use_tc_tiling_on_sc=False),
        scratch_types=[
            pltpu.VMEM((n_ch, _CH), jnp.int32),
            pltpu.VMEM((MC, D), jnp.float32),
            pltpu.SemaphoreType.DMA,
        ],
    )
    def sc_gather(idx_hbm, w_hbm, out_hbm, idx_v, rows_v, gsem):
        wid = lax.axis_index("s") * _NC + lax.axis_index("c")
        base = wid * b_per_w
        pltpu.sync_copy(idx_hbm.at[wid], idx_v)

        @pl.loop(0, NM)
        def _macro(m):
            for j in range(_K):
                pltpu.async_copy(
                    w_hbm.at[idx_v.at[m * _K + j]],
                    rows_v.at[pl.ds(j * _CH, _CH)],
                    gsem)
            for j in range(_K):
                pltpu.make_async_copy(
                    w_hbm.at[idx_v.at[m * _K + j]],
                    rows_v.at[pl.ds(j * _CH, _CH)],
                    gsem).wait()
            pltpu.sync_copy(rows_v, out_hbm.at[pl.ds(base + m * MC, MC)])

    out = sc_gather(idx, weight)
    return out.reshape(B0, S, D)


# speedup vs baseline: 1.8722x; 1.0091x over previous
import functools

import jax
import jax.numpy as jnp
from jax import lax
from jax.experimental import pallas as pl
from jax.experimental.pallas import tpu as pltpu
from jax.experimental.pallas import tpu_sc as plsc

_NC = 2
_NS = 16
_NW = _NC * _NS

_CH = 128
_K = 4


def kernel(token_ids, weight):
    B0, S = token_ids.shape
    V, D = weight.shape
    B = B0 * S
    MC = _CH * _K
    b_per_w = B // _NW
    n_ch = b_per_w // _CH
    NM = b_per_w // MC
    assert B == _NW * n_ch * _CH and b_per_w % MC == 0

    idx = token_ids.reshape(_NW, n_ch, _CH).astype(jnp.int32)

    mesh = plsc.VectorSubcoreMesh(
        core_axis_name="c", subcore_axis_name="s",
        num_cores=_NC, num_subcores=_NS)

    @functools.partial(
        pl.kernel,
        out_type=jax.ShapeDtypeStruct((B, D), jnp.float32),
        mesh=mesh,
        compiler_params=pltpu.CompilerParams(use_tc_tiling_on_sc=False),
        scratch_types=[
            pltpu.VMEM((n_ch, _CH), jnp.int32),
            pltpu.VMEM((2, MC, D), jnp.float32),
            pltpu.SemaphoreType.DMA,
        ],
    )
    def sc_gather(idx_hbm, w_hbm, out_hbm, idx_v, rows_v, gsem):
        wid = lax.axis_index("s") * _NC + lax.axis_index("c")
        base = wid * b_per_w
        pltpu.sync_copy(idx_hbm.at[wid], idx_v)

        def fire(m, slot):
            for j in range(_K):
                pltpu.async_copy(
                    w_hbm.at[idx_v.at[m * _K + j]],
                    rows_v.at[slot, pl.ds(j * _CH, _CH)],
                    gsem)

        def drain(m, slot):
            for j in range(_K):
                pltpu.make_async_copy(
                    w_hbm.at[idx_v.at[m * _K + j]],
                    rows_v.at[slot, pl.ds(j * _CH, _CH)],
                    gsem).wait()

        fire(0, 0)

        @pl.loop(0, NM, step=2)
        def _macro(m):
            for slot in range(2):
                mm = m + slot
                drain(mm, slot)

                @pl.when(mm + 1 < NM)
                def _():
                    fire(mm + 1, 1 - slot)

                pltpu.sync_copy(rows_v.at[slot],
                                out_hbm.at[pl.ds(base + mm * MC, MC)])

    out = sc_gather(idx, weight)
    return out.reshape(B0, S, D)


# speedup vs baseline: 1.8748x; 1.0014x over previous
import functools

import jax
import jax.numpy as jnp
from jax import lax
from jax.experimental import pallas as pl
from jax.experimental.pallas import tpu as pltpu
from jax.experimental.pallas import tpu_sc as plsc

_NC = 2
_NS = 16
_NW = _NC * _NS

_CH = 128
_K = 5


def kernel(token_ids, weight):
    B0, S = token_ids.shape
    V, D = weight.shape
    B = B0 * S
    MC = _CH * _K
    b_per_w = B // _NW
    n_ch = b_per_w // _CH
    NM = b_per_w // MC
    assert B == _NW * n_ch * _CH and b_per_w % MC == 0

    idx = token_ids.reshape(_NW, n_ch, _CH).astype(jnp.int32)

    mesh = plsc.VectorSubcoreMesh(
        core_axis_name="c", subcore_axis_name="s",
        num_cores=_NC, num_subcores=_NS)

    @functools.partial(
        pl.kernel,
        out_type=jax.ShapeDtypeStruct((B, D), jnp.float32),
        mesh=mesh,
        compiler_params=pltpu.CompilerParams(use_tc_tiling_on_sc=False),
        scratch_types=[
            pltpu.VMEM((n_ch, _CH), jnp.int32),
            pltpu.VMEM((2, MC, D), jnp.float32),
            pltpu.SemaphoreType.DMA,
        ],
    )
    def sc_gather(idx_hbm, w_hbm, out_hbm, idx_v, rows_v, gsem):
        wid = lax.axis_index("s") * _NC + lax.axis_index("c")
        base = wid * b_per_w
        pltpu.sync_copy(idx_hbm.at[wid], idx_v)

        def fire(m, slot):
            for j in range(_K):
                pltpu.async_copy(
                    w_hbm.at[idx_v.at[m * _K + j]],
                    rows_v.at[slot, pl.ds(j * _CH, _CH)],
                    gsem)

        def drain(m, slot):
            for j in range(_K):
                pltpu.make_async_copy(
                    w_hbm.at[idx_v.at[m * _K + j]],
                    rows_v.at[slot, pl.ds(j * _CH, _CH)],
                    gsem).wait()

        fire(0, 0)

        @pl.loop(0, NM, step=2)
        def _macro(m):
            for slot in range(2):
                mm = m + slot

                @pl.when(mm + 1 < NM)
                def _():
                    fire(mm + 1, 1 - slot)

                drain(mm, slot)
                pltpu.sync_copy(rows_v.at[slot],
                                out_hbm.at[pl.ds(base + mm * MC, MC)])

    out = sc_gather(idx, weight)
    return out.reshape(B0, S, D)


# speedup vs baseline: 1.8764x; 1.0008x over previous
import functools

import jax
import jax.numpy as jnp
from jax import lax
from jax.experimental import pallas as pl
from jax.experimental.pallas import tpu as pltpu
from jax.experimental.pallas import tpu_sc as plsc

_NC = 2
_NS = 16
_NW = _NC * _NS

_CH = 128
_K = 5


def kernel(token_ids, weight):
    B0, S = token_ids.shape
    V, D = weight.shape
    B = B0 * S
    MC = _CH * _K
    b_per_w = B // _NW
    n_ch = b_per_w // _CH
    NM = b_per_w // MC
    assert B == _NW * n_ch * _CH and b_per_w % MC == 0

    idx = token_ids.reshape(_NW, n_ch, _CH).astype(jnp.int32)

    mesh = plsc.VectorSubcoreMesh(
        core_axis_name="c", subcore_axis_name="s",
        num_cores=_NC, num_subcores=_NS)

    @functools.partial(
        pl.kernel,
        out_type=jax.ShapeDtypeStruct((B, D), jnp.float32),
        mesh=mesh,
        compiler_params=pltpu.CompilerParams(
            use_tc_tiling_on_sc=False, needs_layout_passes=False),
        scratch_types=[
            pltpu.VMEM((n_ch, _CH), jnp.int32),
            pltpu.VMEM((2, MC, D), jnp.float32),
            pltpu.SemaphoreType.DMA,
        ],
    )
    def sc_gather(idx_hbm, w_hbm, out_hbm, idx_v, rows_v, gsem):
        wid = lax.axis_index("s") * _NC + lax.axis_index("c")
        base = wid * b_per_w
        pltpu.sync_copy(idx_hbm.at[wid], idx_v)

        def fire(m, slot):
            for j in range(_K):
                pltpu.async_copy(
                    w_hbm.at[idx_v.at[m * _K + j]],
                    rows_v.at[slot, pl.ds(j * _CH, _CH)],
                    gsem)

        def drain(m, slot):
            for j in range(_K):
                pltpu.make_async_copy(
                    w_hbm.at[idx_v.at[m * _K + j]],
                    rows_v.at[slot, pl.ds(j * _CH, _CH)],
                    gsem).wait()

        fire(0, 0)

        @pl.loop(0, NM, step=2)
        def _macro(m):
            for slot in range(2):
                mm = m + slot

                @pl.when(mm + 1 < NM)
                def _():
                    fire(mm + 1, 1 - slot)

                drain(mm, slot)
                pltpu.sync_copy(rows_v.at[slot],
                                out_hbm.at[pl.ds(base + mm * MC, MC)])

    out = sc_gather(idx, weight)
    return out.reshape(B0, S, D)


# speedup vs baseline: 2.0030x; 1.0675x over previous
import functools

import jax
import jax.numpy as jnp
from jax import lax
from jax.experimental import pallas as pl
from jax.experimental.pallas import tpu as pltpu
from jax.experimental.pallas import tpu_sc as plsc

_NC = 2
_NS = 16
_NW = _NC * _NS
_L = 16

_T = 128


def kernel(token_ids, weight):
    B0, S = token_ids.shape
    V, D = weight.shape
    NBLK = B0 // _T
    G = S * NBLK
    g_per_w = G // _NW
    DB = D // 8
    assert G == g_per_w * _NW and g_per_w % 2 == 0 and B0 % _T == 0 and D % 8 == 0

    idx2 = token_ids.T.reshape(G, _T).astype(jnp.int32)

    mesh = plsc.VectorSubcoreMesh(
        core_axis_name="c", subcore_axis_name="s",
        num_cores=_NC, num_subcores=_NS)

    @functools.partial(
        pl.kernel,
        out_type=jax.ShapeDtypeStruct((S, DB, NBLK, 8 * _T), jnp.float32),
        mesh=mesh,
        compiler_params=pltpu.CompilerParams(
            use_tc_tiling_on_sc=False, needs_layout_passes=False),
        scratch_types=[
            pltpu.VMEM((g_per_w, _T), jnp.int32),
            pltpu.VMEM((_T, D), jnp.float32),
            pltpu.VMEM((_T, D), jnp.float32),
            pltpu.VMEM((D * _T,), jnp.float32),
            pltpu.VMEM((D * _T,), jnp.float32),
            pltpu.SemaphoreType.DMA,
            pltpu.SemaphoreType.DMA,
        ],
    )
    def sc_gather(idx_hbm, w_hbm, out_hbm, idx_v,
                  rows_0, rows_1, tr_0, tr_1, gsem, osem):
        rows_v = (rows_0, rows_1)
        tr_v = (tr_0, tr_1)
        wid = lax.axis_index("s") * _NC + lax.axis_index("c")
        g0 = wid * g_per_w
        pltpu.sync_copy(idx_hbm.at[pl.ds(g0, g_per_w)], idx_v)

        lane = lax.iota(jnp.int32, _L)

        def fire_gather(j, slot):
            pltpu.async_copy(w_hbm.at[idx_v.at[j]], rows_v[slot], gsem)

        def drain_gather(j, slot):
            pltpu.make_async_copy(
                w_hbm.at[idx_v.at[j]], rows_v[slot], gsem).wait()

        def out_descr(j, slot, dblk):
            g = g0 + j
            s = g // NBLK
            bblk = lax.rem(g, NBLK)
            return pltpu.make_async_copy(
                tr_v[slot].at[pl.ds(dblk * 8 * _T, 8 * _T)],
                out_hbm.at[s, dblk, bblk], osem)

        def transpose(slot):
            @pl.loop(0, _L)
            def _diag(k):
                rot = jnp.bitwise_and(lane + k, _L - 1)
                out_k = rot * _T + lane
                for bi0 in range(_T // _L):
                    row = lane + bi0 * _L
                    for dc in range(D // _L):
                        vals = plsc.load_gather(
                            rows_v[slot], [row, rot + dc * _L])
                        plsc.store_scatter(
                            tr_v[slot],
                            [out_k + (dc * _L * _T + bi0 * _L)], vals)

        fire_gather(0, 0)

        @pl.loop(0, g_per_w, step=2)
        def _blk(m):
            for slot in range(2):
                j = m + slot

                @pl.when(j + 1 < g_per_w)
                def _():
                    fire_gather(j + 1, 1 - slot)

                drain_gather(j, slot)

                @pl.when(j >= 2)
                def _():
                    for dblk in range(DB):
                        out_descr(j - 2, slot, dblk).wait()

                transpose(slot)
                for dblk in range(DB):
                    out_descr(j, slot, dblk).start()

        for last in (g_per_w - 2, g_per_w - 1):
            for dblk in range(DB):
                out_descr(last, last % 2, dblk).wait()

    out4 = sc_gather(idx2, weight)
    out5 = out4.reshape(S, DB, NBLK, 8, _T)
    return out5.transpose(2, 4, 0, 1, 3).reshape(B0, S, D)
